# baseline (speedup 1.0000x reference)
import jax
import jax.numpy as jnp
from jax import lax
from jax.experimental import pallas as pl
from jax.experimental.pallas import tpu as pltpu
from jax.experimental.pallas import tpu_sc as plsc

VOCAB = 49408
HIDDEN = 768
SEQ = 77
BATCH = 1024

NC = 2
NS = 16
NW = NC * NS

ROWS = BATCH * SEQ
RPW = ROWS // NW
R = 22
NBUF = 4
A = 3
NCH = RPW // R
G = HIDDEN // 16

assert RPW % R == 0 and NCH % NBUF == 0 and A <= NBUF


def _body(x_hbm, tok_hbm, pos_hbm, out_hbm, idx_v, pos_v, *rest):
  bufs = rest[:NBUF]
  sem_g, sem_s = rest[NBUF], rest[NBUF + 1]
  wid = lax.axis_index("s") * NC + lax.axis_index("c")
  base = wid * RPW

  pltpu.sync_copy(x_hbm.at[wid], idx_v)
  pltpu.sync_copy(pos_hbm, pos_v)

  def gather_start(c, b):
    pltpu.async_copy(tok_hbm.at[idx_v.at[c]], bufs[b], sem_g.at[b])

  def gather_wait(b):
    pltpu.make_async_copy(tok_hbm.at[pl.ds(0, R)], bufs[b], sem_g.at[b]).wait()

  def store_start(c, b):
    pltpu.async_copy(bufs[b], out_hbm.at[pl.ds(base + c * R, R)], sem_s.at[b])

  def store_wait(b):
    pltpu.make_async_copy(
        bufs[b], out_hbm.at[pl.ds(base, R)], sem_s.at[b]).wait()

  for c0 in range(A):
    gather_start(c0, c0 % NBUF)

  @pl.loop(0, NCH, step=NBUF)
  def _outer(g):
    for b in range(NBUF):
      c = g + b
      gather_wait(b)

      off = lax.rem(c * R, SEQ)

      @pl.loop(0, R)
      def _row(r):
        p = off + r
        p = jnp.where(p >= SEQ, p - SEQ, p)
        for h in range(G):
          sl = pl.ds(h * 16, 16)
          plsc.addupdate(bufs[b].at[r, sl], pos_v[p, sl])

      store_start(c, b)
      fb = (b + A) % NBUF

      @pl.when(c + A < NCH)
      def _fire():
        @pl.when(c >= NBUF - A)
        def _drain():
          store_wait(fb)

        gather_start(c + A, fb)

  for b in range(NBUF):
    store_wait(b)


@jax.jit
def kernel(x, token_embedding, position_embedding):
  xr = x.astype(jnp.int32).reshape(NW, NCH, R)
  mesh = plsc.VectorSubcoreMesh(
      core_axis_name="c", subcore_axis_name="s",
      num_cores=NC, num_subcores=NS)
  fn = pl.kernel(
      _body,
      out_type=jax.ShapeDtypeStruct((ROWS, HIDDEN), jnp.float32),
      mesh=mesh,
      scratch_types=(
          [pltpu.VMEM((NCH, R), jnp.int32),
           pltpu.VMEM((SEQ, HIDDEN), jnp.float32)]
          + [pltpu.VMEM((R, HIDDEN), jnp.float32) for _ in range(NBUF)]
          + [pltpu.SemaphoreType.DMA((NBUF,)),
             pltpu.SemaphoreType.DMA((NBUF,))]
      ),
      compiler_params=pltpu.CompilerParams(use_tc_tiling_on_sc=False),
  )
  out = fn(xr, token_embedding, position_embedding)
  return out.reshape(BATCH, SEQ, HIDDEN)


# speedup vs baseline: 1.0006x; 1.0006x over previous
import jax
import jax.numpy as jnp
from jax import lax
from jax.experimental import pallas as pl
from jax.experimental.pallas import tpu as pltpu
from jax.experimental.pallas import tpu_sc as plsc

VOCAB = 49408
HIDDEN = 768
SEQ = 77
BATCH = 1024

NC = 2
NS = 16
NW = NC * NS

ROWS = BATCH * SEQ
RPW = ROWS // NW
R = 22
NBUF = 4
A = 2
NCH = RPW // R
G = HIDDEN // 16

assert RPW % R == 0 and NCH % NBUF == 0 and A <= NBUF


def _body(x_hbm, tok_hbm, pos_hbm, out_hbm, idx_v, pos_v, *rest):
  bufs = rest[:NBUF]
  sem_g, sem_s = rest[NBUF], rest[NBUF + 1]
  wid = lax.axis_index("s") * NC + lax.axis_index("c")
  base = wid * RPW

  pltpu.sync_copy(x_hbm.at[wid], idx_v)
  pltpu.sync_copy(pos_hbm, pos_v)

  def gather_start(c, b):
    pltpu.async_copy(tok_hbm.at[idx_v.at[c]], bufs[b], sem_g.at[b])

  def gather_wait(b):
    pltpu.make_async_copy(tok_hbm.at[pl.ds(0, R)], bufs[b], sem_g.at[b]).wait()

  def store_start(c, b):
    pltpu.async_copy(bufs[b], out_hbm.at[pl.ds(base + c * R, R)], sem_s.at[b])

  def store_wait(b):
    pltpu.make_async_copy(
        bufs[b], out_hbm.at[pl.ds(base, R)], sem_s.at[b]).wait()

  for c0 in range(A):
    gather_start(c0, c0 % NBUF)

  @pl.loop(0, NCH, step=NBUF)
  def _outer(g):
    for b in range(NBUF):
      c = g + b
      gather_wait(b)

      fb = (b + A) % NBUF

      @pl.when(c + A < NCH)
      def _fire():
        @pl.when(c >= NBUF - A)
        def _drain():
          store_wait(fb)

        gather_start(c + A, fb)

      off = lax.rem(c * R, SEQ)

      @pl.loop(0, R)
      def _row(r):
        p = off + r
        p = jnp.where(p >= SEQ, p - SEQ, p)
        for h in range(G):
          sl = pl.ds(h * 16, 16)
          plsc.addupdate(bufs[b].at[r, sl], pos_v[p, sl])

      store_start(c, b)

  for b in range(NBUF):
    store_wait(b)


@jax.jit
def kernel(x, token_embedding, position_embedding):
  xr = x.astype(jnp.int32).reshape(NW, NCH, R)
  mesh = plsc.VectorSubcoreMesh(
      core_axis_name="c", subcore_axis_name="s",
      num_cores=NC, num_subcores=NS)
  fn = pl.kernel(
      _body,
      out_type=jax.ShapeDtypeStruct((ROWS, HIDDEN), jnp.float32),
      mesh=mesh,
      scratch_types=(
          [pltpu.VMEM((NCH, R), jnp.int32),
           pltpu.VMEM((SEQ, HIDDEN), jnp.float32)]
          + [pltpu.VMEM((R, HIDDEN), jnp.float32) for _ in range(NBUF)]
          + [pltpu.SemaphoreType.DMA((NBUF,)),
             pltpu.SemaphoreType.DMA((NBUF,))]
      ),
      compiler_params=pltpu.CompilerParams(use_tc_tiling_on_sc=False),
  )
  out = fn(xr, token_embedding, position_embedding)
  return out.reshape(BATCH, SEQ, HIDDEN)


# speedup vs baseline: 1.3031x; 1.3022x over previous
import jax
import jax.numpy as jnp
from jax import lax
from jax.experimental import pallas as pl
from jax.experimental.pallas import tpu as pltpu
from jax.experimental.pallas import tpu_sc as plsc

VOCAB = 49408
HIDDEN = 768
SEQ = 77
BATCH = 1024

NC = 2
NS = 16
NW = NC * NS

ROWS = BATCH * SEQ
RPW = ROWS // NW
R = 22
NBUF = 4
A = 2
NCH = RPW // R
G = HIDDEN // 16

assert RPW % R == 0 and NCH % NBUF == 0 and A <= NBUF


def _body(x_hbm, tok_hbm, pos_hbm, out_hbm, idx_v, pos_v, *rest):
  bufs = rest[:NBUF]
  sem_g, sem_s = rest[NBUF], rest[NBUF + 1]
  wid = lax.axis_index("s") * NC + lax.axis_index("c")
  base = wid * RPW

  pltpu.sync_copy(x_hbm.at[wid], idx_v)
  pltpu.sync_copy(pos_hbm, pos_v)

  def gather_start(c, b):
    pltpu.async_copy(tok_hbm.at[idx_v.at[c]], bufs[b], sem_g.at[b])

  def gather_wait(b):
    pltpu.make_async_copy(tok_hbm.at[pl.ds(0, R)], bufs[b], sem_g.at[b]).wait()

  def store_start(c, b):
    pltpu.async_copy(bufs[b], out_hbm.at[pl.ds(base + c * R, R)], sem_s.at[b])

  def store_wait(b):
    pltpu.make_async_copy(
        bufs[b], out_hbm.at[pl.ds(base, R)], sem_s.at[b]).wait()

  for c0 in range(A):
    gather_start(c0, c0 % NBUF)

  @pl.loop(0, NCH, step=NBUF)
  def _outer(g):
    for b in range(NBUF):
      c = g + b
      gather_wait(b)

      fb = (b + A) % NBUF

      @pl.when(c + A < NCH)
      def _fire():
        @pl.when(c >= NBUF - A)
        def _drain():
          store_wait(fb)

        gather_start(c + A, fb)

      off = lax.rem(c * R, SEQ)

      @plsc.parallel_loop(0, R, 1, unroll=2)
      def _row(r):
        p = off + r
        p = jnp.where(p >= SEQ, p - SEQ, p)
        for h in range(G):
          sl = pl.ds(h * 16, 16)
          plsc.addupdate(bufs[b].at[r, sl], pos_v[p, sl])

      store_start(c, b)

  for b in range(NBUF):
    store_wait(b)


@jax.jit
def kernel(x, token_embedding, position_embedding):
  xr = x.astype(jnp.int32).reshape(NW, NCH, R)
  mesh = plsc.VectorSubcoreMesh(
      core_axis_name="c", subcore_axis_name="s",
      num_cores=NC, num_subcores=NS)
  fn = pl.kernel(
      _body,
      out_type=jax.ShapeDtypeStruct((ROWS, HIDDEN), jnp.float32),
      mesh=mesh,
      scratch_types=(
          [pltpu.VMEM((NCH, R), jnp.int32),
           pltpu.VMEM((SEQ, HIDDEN), jnp.float32)]
          + [pltpu.VMEM((R, HIDDEN), jnp.float32) for _ in range(NBUF)]
          + [pltpu.SemaphoreType.DMA((NBUF,)),
             pltpu.SemaphoreType.DMA((NBUF,))]
      ),
      compiler_params=pltpu.CompilerParams(use_tc_tiling_on_sc=False),
  )
  out = fn(xr, token_embedding, position_embedding)
  return out.reshape(BATCH, SEQ, HIDDEN)
